# baseline (speedup 1.0000x reference)
import functools

import jax
import jax.numpy as jnp
from jax import lax
from jax.experimental import pallas as pl
from jax.experimental.pallas import tpu as pltpu
from jax.experimental.pallas import tpu_sc as plsc

_NC = 2
_NS = 16
_NW = _NC * _NS
_B = 16384
_D = 64
_BPW = _B // _NW
_CH = 128
_NCH = _BPW // _CH
_EPS = 1e-6


def _vsqrt(x):
    xs = jnp.maximum(x, jnp.float32(1e-35))
    i = lax.bitcast_convert_type(xs, jnp.int32)
    i = jnp.int32(0x5F3759DF) - lax.shift_right_logical(i, 1)
    y = lax.bitcast_convert_type(i, jnp.float32)
    for _ in range(3):
        y = y * (jnp.float32(1.5) - jnp.float32(0.5) * xs * y * y)
    return xs * y


_mesh = plsc.VectorSubcoreMesh(core_axis_name="c", subcore_axis_name="s")


@functools.partial(
    pl.kernel,
    out_type=jax.ShapeDtypeStruct((_NW, 16), jnp.float32),
    mesh=_mesh,
    compiler_params=pltpu.CompilerParams(needs_layout_passes=False),
    scratch_types=[
        pltpu.VMEM((_BPW,), jnp.int32),
        pltpu.VMEM((_BPW,), jnp.int32),
        pltpu.VMEM((_CH, 2 * _D), jnp.float32),
        pltpu.VMEM((_CH, 2 * _D), jnp.float32),
        pltpu.VMEM((_CH, _D), jnp.float32),
        pltpu.VMEM((_CH, _D), jnp.float32),
        pltpu.VMEM((16,), jnp.float32),
        pltpu.SemaphoreType.DMA,
        pltpu.SemaphoreType.DMA,
    ],
)
def _center_loss_sc(table2, labels, batch, out, lab_v, widx_v, rows0_v,
                    rows1_v, batch0_v, batch1_v, acc_v, sem0, sem1):
    wid = lax.axis_index("s") * _NC + lax.axis_index("c")

    pltpu.sync_copy(labels.at[pl.ds(wid * _BPW, _BPW)], lab_v)
    for j in range(_BPW // 16):
        widx_v[pl.ds(j * 16, 16)] = lax.shift_right_logical(
            lab_v[pl.ds(j * 16, 16)], 1)

    rows_slots = (rows0_v, rows1_v)
    batch_slots = (batch0_v, batch1_v)
    sems = (sem0, sem1)

    def fire(j):
        rv, bv, sm = rows_slots[j % 2], batch_slots[j % 2], sems[j % 2]
        return [
            pltpu.async_copy(table2.at[widx_v.at[pl.ds(j * _CH, _CH)]],
                             rv, sm),
            pltpu.async_copy(
                batch.at[pl.ds(wid * _BPW + j * _CH, _CH)], bv, sm),
        ]

    lanes = lax.iota(jnp.int32, 16)

    def compute_chunk(j, acc):
        rv, bv = rows_slots[j % 2], batch_slots[j % 2]

        def blk_body(blk, a):
            labv = lab_v[pl.ds(j * _CH + blk * 16, 16)]
            half = lax.shift_left(jnp.bitwise_and(labv, 1), 6)
            rows = blk * 16 + lanes
            tot = jnp.zeros((16,), jnp.float32)
            for f in range(_D):
                g = plsc.load_gather(rv, [rows, half + f])
                b = plsc.load_gather(
                    bv, [rows, jnp.full((16,), f, jnp.int32)])
                d = g - b + jnp.float32(_EPS)
                tot = tot + d * d
            return a + _vsqrt(tot)

        return lax.fori_loop(0, _CH // 16, blk_body, acc)

    acc = jnp.zeros((16,), jnp.float32)
    inflight = fire(0)
    for j in range(_NCH):
        nxt = fire(j + 1) if j + 1 < _NCH else []
        for cp in inflight:
            cp.wait()
        inflight = nxt
        acc = compute_chunk(j, acc)

    acc_v[...] = acc
    pltpu.sync_copy(acc_v, out.at[wid])


def kernel(center_list, batch_center_vecs, gt_labels):
    table2 = center_list.reshape(500000, 2 * _D)
    partials = _center_loss_sc(table2, gt_labels, batch_center_vecs)
    return jnp.sum(partials) / jnp.float32(_B)


# speedup vs baseline: 2.2444x; 2.2444x over previous
import functools

import jax
import jax.numpy as jnp
from jax import lax
from jax.experimental import pallas as pl
from jax.experimental.pallas import tpu as pltpu
from jax.experimental.pallas import tpu_sc as plsc

_NC = 2
_NS = 16
_NW = _NC * _NS
_B = 16384
_D = 64
_BPW = _B // _NW
_NSLOT = 16
_EPS = 1e-6


def _vsqrt(x):
    xs = jnp.maximum(x, jnp.float32(1e-35))
    i = lax.bitcast_convert_type(xs, jnp.int32)
    i = jnp.int32(0x5F3759DF) - lax.shift_right_logical(i, 1)
    y = lax.bitcast_convert_type(i, jnp.float32)
    for _ in range(3):
        y = y * (jnp.float32(1.5) - jnp.float32(0.5) * xs * y * y)
    return xs * y


_mesh = plsc.VectorSubcoreMesh(core_axis_name="c", subcore_axis_name="s")


@functools.partial(
    pl.kernel,
    out_type=jax.ShapeDtypeStruct((_NW, 16), jnp.float32),
    mesh=_mesh,
    compiler_params=pltpu.CompilerParams(needs_layout_passes=False),
    scratch_types=[
        pltpu.VMEM((_BPW,), jnp.int32),
        pltpu.VMEM((_NSLOT * 8, _D), jnp.float32),
        pltpu.VMEM((_BPW, _D), jnp.float32),
        pltpu.VMEM((16,), jnp.float32),
        [pltpu.SemaphoreType.DMA] * _NSLOT,
        pltpu.SemaphoreType.DMA,
    ],
)
def _center_loss_sc(table, labels, batch, out, lab_v, ring_v, batch_v,
                    acc_v, sems, semb):
    wid = lax.axis_index("s") * _NC + lax.axis_index("c")

    cpb = pltpu.async_copy(batch.at[pl.ds(wid * _BPW, _BPW)], batch_v, semb)
    pltpu.sync_copy(labels.at[pl.ds(wid * _BPW, _BPW)], lab_v)
    cpb.wait()

    lanes = lax.iota(jnp.int32, 16)

    def fire(labv, i, slot):
        gid = lax.shift_right_logical(labv[i], 3)
        return pltpu.async_copy(
            table.at[gid],
            ring_v.at[pl.ds(slot * 8, 8)],
            sems[slot])

    labv0 = lab_v[pl.ds(0, 16)]
    inflight = [fire(labv0, i, i) for i in range(_NSLOT)]

    def blk_body(blk, acc):
        labv = lab_v[pl.ds(blk * 16, 16)]
        nxtv = lab_v[pl.ds(jnp.minimum(blk + 1, _BPW // 16 - 1) * 16, 16)]
        merged = jnp.zeros((16,), jnp.float32)
        for i in range(16):
            slot = i
            inflight[slot].wait()
            sub = jnp.bitwise_and(labv[i], 7)
            base = slot * 8 + sub
            sq = None
            for k in range(_D // 16):
                g = ring_v[base, pl.ds(k * 16, 16)]
                b = batch_v[blk * 16 + i, pl.ds(k * 16, 16)]
                d = g - b + jnp.float32(_EPS)
                m = d * d
                sq = m if sq is None else sq + m
            tot = jnp.sum(sq)
            merged = jnp.where(lanes == i, tot, merged)
            inflight[slot] = fire(nxtv, i, slot)
        return acc + _vsqrt(merged)

    acc = lax.fori_loop(0, _BPW // 16, blk_body,
                        jnp.zeros((16,), jnp.float32))
    for slot in range(_NSLOT):
        pltpu.make_async_copy(
            table.at[0], ring_v.at[pl.ds(slot * 8, 8)], sems[slot]).wait()

    acc_v[...] = acc
    pltpu.sync_copy(acc_v, out.at[wid])


def kernel(center_list, batch_center_vecs, gt_labels):
    table3 = center_list.reshape(125000, 8, _D)
    partials = _center_loss_sc(table3, gt_labels, batch_center_vecs)
    return jnp.sum(partials) / jnp.float32(_B)
